# Initial kernel scaffold; baseline (speedup 1.0000x reference)
import jax
import jax.numpy as jnp
from jax.experimental import pallas as pl


def kernel(x, table):
    raise NotImplementedError("write your pallas kernel here")


# speedup vs baseline: 12.0917x; 12.0917x over previous
import jax
import jax.numpy as jnp
from jax.experimental import pallas as pl


def _broadcast_body(t_ref, o_ref):
    o_ref[...] = jnp.broadcast_to(t_ref[...], o_ref.shape)


def kernel(x, table):
    batch, seq_len = x.shape
    _, dim = table.shape
    width = seq_len * dim
    flat = table[:seq_len].reshape(1, width)

    bb = 64
    out = pl.pallas_call(
        _broadcast_body,
        grid=(batch // bb,),
        in_specs=[pl.BlockSpec((1, width), lambda i: (0, 0))],
        out_specs=pl.BlockSpec((bb, width), lambda i: (i, 0)),
        out_shape=jax.ShapeDtypeStruct((batch, width), jnp.float32),
    )(flat)
    return out.reshape(batch, seq_len, dim)


# speedup vs baseline: 12.1073x; 1.0013x over previous
import jax
import jax.numpy as jnp
from jax.experimental import pallas as pl


def _broadcast_body(t_ref, o_ref):
    o_ref[...] = jnp.broadcast_to(t_ref[...], o_ref.shape)


def kernel(x, table):
    batch, seq_len = x.shape
    _, dim = table.shape
    width = seq_len * dim
    flat = table[:seq_len].reshape(1, width)

    bb = 256
    out = pl.pallas_call(
        _broadcast_body,
        grid=(batch // bb,),
        in_specs=[pl.BlockSpec((1, width), lambda i: (0, 0))],
        out_specs=pl.BlockSpec((bb, width), lambda i: (i, 0)),
        out_shape=jax.ShapeDtypeStruct((batch, width), jnp.float32),
    )(flat)
    return out.reshape(batch, seq_len, dim)
